# Initial kernel scaffold; baseline (speedup 1.0000x reference)
import jax
import jax.numpy as jnp
from jax.experimental import pallas as pl


def kernel(input_data, target_data, mask_src, mask_tar, index, ref):
    raise NotImplementedError("write your pallas kernel here")


# speedup vs baseline: 276.1421x; 276.1421x over previous
import functools

import jax
import jax.numpy as jnp
from jax import lax
from jax.experimental import pallas as pl
from jax.experimental.pallas import tpu as pltpu
from jax.experimental.pallas import tpu_sc as plsc

H = 512
P = H * H
N = 65536
NC, NS, L = 2, 16, 16
NW = NC * NS
CHUNK = N // NW
NV = CHUNK // L
NB = 256
NSC = 6
HISTW = NSC * NB


def _dn255(x):
    return jnp.clip((x + 1.0) / 2.0, 0.0, 1.0) * 255.0


def _sc_gather_hist(rf0, rf1, rf2, ms, tg0, tg1, tg2, mt, in0, in1, in2,
                    lin_r, lin_t, zer,
                    hist_out, counts_out, ints_out, rm_out, im_out,
                    idxr_v, idxt_v, g_v, hist16_v, histred_v, ones_v,
                    ints_v, cbuf_v, counts_sh, sem):
    cid = lax.axis_index("c")
    sid = lax.axis_index("s")
    wid = sid * NC + cid
    base = wid * CHUNK

    pltpu.sync_copy(lin_r.at[pl.ds(base, CHUNK)], idxr_v)
    pltpu.sync_copy(lin_t.at[pl.ds(base, CHUNK)], idxt_v)

    seg = P // NS
    pltpu.sync_copy(zer.at[pl.ds(sid * seg, seg)],
                    counts_sh.at[pl.ds(sid * seg, seg)])

    srcs_r = (rf0, rf1, rf2, ms, in0, in1, in2)
    srcs_t = (tg0, tg1, tg2, mt)
    descs = []
    for i, s in enumerate(srcs_r):
        descs.append(pltpu.async_copy(
            s.at[idxr_v], g_v.at[pl.ds(i * CHUNK, CHUNK)], sem))
    for i, s in enumerate(srcs_t):
        descs.append(pltpu.async_copy(
            s.at[idxt_v], g_v.at[pl.ds((7 + i) * CHUNK, CHUNK)], sem))

    zero16 = jnp.zeros((L,), jnp.int32)
    one16 = jnp.full((L,), 1, jnp.int32)

    def zstep(j, _):
        hist16_v[pl.ds(j * L, L)] = zero16
        return 0
    lax.fori_loop(0, (L * HISTW) // L, zstep, 0)

    def ostep(j, _):
        ones_v[pl.ds(j * L, L)] = one16
        return 0
    lax.fori_loop(0, NV, ostep, 0)

    for d in descs:
        d.wait()

    plsc.subcore_barrier()
    pltpu.sync_copy(ones_v, counts_sh.at[idxr_v], add=True)

    laneoff = lax.iota(jnp.int32, L) * HISTW

    def step(i, _):
        off = i * L
        msv = g_v[pl.ds(3 * CHUNK + off, L)]
        mtv = g_v[pl.ds(10 * CHUNK + off, L)]
        for c in range(3):
            rm = _dn255(g_v[pl.ds(c * CHUNK + off, L)]) * msv
            im = _dn255(g_v[pl.ds((4 + c) * CHUNK + off, L)]) * msv
            tm = _dn255(g_v[pl.ds((7 + c) * CHUNK + off, L)]) * mtv
            b_r = jnp.clip(rm, 0.0, 255.0).astype(jnp.int32)
            b_t = jnp.clip(tm, 0.0, 255.0).astype(jnp.int32)
            plsc.addupdate_scatter(hist16_v, [laneoff + (c * NB + b_r)],
                                   one16)
            plsc.addupdate_scatter(hist16_v, [laneoff + ((3 + c) * NB + b_t)],
                                   one16)
            g_v[pl.ds(c * CHUNK + off, L)] = rm
            g_v[pl.ds((4 + c) * CHUNK + off, L)] = im
            ints_v[pl.ds(c * CHUNK + off, L)] = b_r
        return 0
    lax.fori_loop(0, NV, step, 0)

    def rstep(j, _):
        acc = hist16_v[pl.ds(j * L, L)]
        for l in range(1, L):
            acc = acc + hist16_v[pl.ds(l * HISTW + j * L, L)]
        histred_v[pl.ds(j * L, L)] = acc
        return 0
    lax.fori_loop(0, HISTW // L, rstep, 0)

    pltpu.sync_copy(histred_v, hist_out.at[pl.ds(wid * HISTW, HISTW)])
    for c in range(3):
        pltpu.sync_copy(g_v.at[pl.ds(c * CHUNK, CHUNK)],
                        rm_out.at[pl.ds(c * N + base, CHUNK)])
        pltpu.sync_copy(g_v.at[pl.ds((4 + c) * CHUNK, CHUNK)],
                        im_out.at[pl.ds(c * N + base, CHUNK)])
        pltpu.sync_copy(ints_v.at[pl.ds(c * CHUNK, CHUNK)],
                        ints_out.at[pl.ds(c * N + base, CHUNK)])

    plsc.subcore_barrier()
    pltpu.sync_copy(counts_sh.at[pl.ds(sid * seg, seg)], cbuf_v)
    pltpu.sync_copy(cbuf_v, counts_out.at[pl.ds(cid * P + sid * seg, seg)])


def _tc_tables_base(inp_ref, ref_ref, ms_ref, hist_ref, cnt_ref,
                    tab_ref, inv_ref, s1_ref):
    g = pl.program_id(0)

    @pl.when(g == 0)
    def _():
        hist6 = jnp.sum(hist_ref[...], axis=0).reshape(NSC, NB)
        hist6 = hist6.astype(jnp.float32)
        row = lax.broadcasted_iota(jnp.int32, (NB, NB), 0)
        col = lax.broadcasted_iota(jnp.int32, (NB, NB), 1)
        tri = (row <= col).astype(jnp.float32)
        cdf = jax.lax.dot(hist6, tri,
                          preferred_element_type=jnp.float32) * (1.0 / N)
        rcdf = cdf[0:3]
        acdf = cdf[3:6]
        cmp = (acdf[:, None, :] < rcdf[:, :, None]).astype(jnp.float32)
        t = jnp.sum(cmp, axis=2)
        ii = lax.broadcasted_iota(jnp.int32, (3, NB), 1).astype(jnp.float32)
        tab = jnp.where(t >= 1.0, t,
                        jnp.where(rcdf >= acdf[:, 0:1], 1.0, ii))
        tab = jnp.where(ii == 0.0, 0.0, jnp.where(ii == 255.0, 255.0, tab))
        tab_ref[...] = tab

    cnt = cnt_ref[0] + cnt_ref[1]
    inv_ref[...] = 1.0 / cnt.astype(jnp.float32)

    msv = ms_ref[...][None]
    im = _dn255(inp_ref[...]) * msv
    rm = _dn255(ref_ref[...]) * msv
    part = jnp.sum(jnp.abs(im - rm)).reshape(1, 1)

    @pl.when(g == 0)
    def _():
        s1_ref[...] = part

    @pl.when(g > 0)
    def _():
        s1_ref[...] += part


def _sc_correction(tab, inv_img, lin_r, ints, rm, im,
                   cpart_out,
                   idxr_v, invc_v, tab_v, intsb_v, rmb_v, imb_v, acc_v, sem):
    cid = lax.axis_index("c")
    sid = lax.axis_index("s")
    wid = sid * NC + cid
    base = wid * CHUNK

    pltpu.sync_copy(lin_r.at[pl.ds(base, CHUNK)], idxr_v)
    d = pltpu.async_copy(inv_img.at[idxr_v], invc_v, sem)
    pltpu.sync_copy(tab, tab_v)
    for c in range(3):
        pltpu.sync_copy(ints.at[pl.ds(c * N + base, CHUNK)],
                        intsb_v.at[pl.ds(c * CHUNK, CHUNK)])
        pltpu.sync_copy(rm.at[pl.ds(c * N + base, CHUNK)],
                        rmb_v.at[pl.ds(c * CHUNK, CHUNK)])
        pltpu.sync_copy(im.at[pl.ds(c * N + base, CHUNK)],
                        imb_v.at[pl.ds(c * CHUNK, CHUNK)])
    d.wait()

    def step(i, acc):
        off = i * L
        invv = invc_v[pl.ds(off, L)]
        for c in range(3):
            b = intsb_v[pl.ds(c * CHUNK + off, L)]
            v = plsc.load_gather(tab_v, [b + (c * NB)])
            imv = imb_v[pl.ds(c * CHUNK + off, L)]
            rmv = rmb_v[pl.ds(c * CHUNK + off, L)]
            dlt = jnp.abs(imv - v) - jnp.abs(imv - rmv)
            acc = acc + dlt * invv
        return acc
    acc = lax.fori_loop(0, NV, step, jnp.zeros((L,), jnp.float32))
    acc_v[...] = acc
    pltpu.sync_copy(acc_v, cpart_out.at[pl.ds(wid * L, L)])


def _tc_finish(s1_ref, cpart_ref, out_ref):
    tot = s1_ref[...] + jnp.sum(cpart_ref[...]).reshape(1, 1)
    out_ref[...] = tot * (1.0 / (3.0 * P))


def kernel(input_data, target_data, mask_src, mask_tar, index, ref):
    inp3 = input_data.reshape(3, H, H)
    tgt3 = target_data.reshape(3, H, H)
    rf3 = ref.reshape(3, H, H)
    ms2 = mask_src.reshape(H, H)
    mt2 = mask_tar.reshape(H, H)

    inpf = inp3.reshape(3, P)
    tgtf = tgt3.reshape(3, P)
    rff = rf3.reshape(3, P)
    msf = ms2.reshape(P)
    mtf = mt2.reshape(P)

    lin_r = (index[0, 0] * H + index[1, 0]).astype(jnp.int32)
    lin_t = (index[2, 0] * H + index[3, 0]).astype(jnp.int32)
    zer = jnp.zeros((P,), jnp.int32)

    mesh = plsc.VectorSubcoreMesh(core_axis_name="c", subcore_axis_name="s",
                                  num_cores=NC, num_subcores=NS)

    sc_params = pltpu.CompilerParams(needs_layout_passes=False)
    k1 = pl.kernel(
        _sc_gather_hist,
        out_type=(
            jax.ShapeDtypeStruct((NW * HISTW,), jnp.int32),
            jax.ShapeDtypeStruct((NC * P,), jnp.int32),
            jax.ShapeDtypeStruct((3 * N,), jnp.int32),
            jax.ShapeDtypeStruct((3 * N,), jnp.float32),
            jax.ShapeDtypeStruct((3 * N,), jnp.float32),
        ),
        mesh=mesh,
        compiler_params=sc_params,
        scratch_types=[
            pltpu.VMEM((CHUNK,), jnp.int32),
            pltpu.VMEM((CHUNK,), jnp.int32),
            pltpu.VMEM((11 * CHUNK,), jnp.float32),
            pltpu.VMEM((L * HISTW,), jnp.int32),
            pltpu.VMEM((HISTW,), jnp.int32),
            pltpu.VMEM((CHUNK,), jnp.int32),
            pltpu.VMEM((3 * CHUNK,), jnp.int32),
            pltpu.VMEM((P // NS,), jnp.int32),
            pltpu.VMEM_SHARED((P,), jnp.int32),
            pltpu.SemaphoreType.DMA,
        ],
    )
    hist, counts, ints, rm, im = k1(rff[0], rff[1], rff[2], msf,
                                    tgtf[0], tgtf[1], tgtf[2], mtf,
                                    inpf[0], inpf[1], inpf[2],
                                    lin_r, lin_t, zer)

    counts3 = counts.reshape(2, H, H)
    hist2 = hist.reshape(NW, HISTW)
    grid_r = 8
    rows = H // grid_r
    tab, inv_img, s1 = pl.pallas_call(
        _tc_tables_base,
        grid=(grid_r,),
        in_specs=[
            pl.BlockSpec((3, rows, H), lambda g: (0, g, 0)),
            pl.BlockSpec((3, rows, H), lambda g: (0, g, 0)),
            pl.BlockSpec((rows, H), lambda g: (g, 0)),
            pl.BlockSpec((NW, HISTW), lambda g: (0, 0)),
            pl.BlockSpec((2, rows, H), lambda g: (0, g, 0)),
        ],
        out_specs=[
            pl.BlockSpec((3, NB), lambda g: (0, 0)),
            pl.BlockSpec((rows, H), lambda g: (g, 0)),
            pl.BlockSpec((1, 1), lambda g: (0, 0)),
        ],
        out_shape=[
            jax.ShapeDtypeStruct((3, NB), jnp.float32),
            jax.ShapeDtypeStruct((H, H), jnp.float32),
            jax.ShapeDtypeStruct((1, 1), jnp.float32),
        ],
    )(inp3, rf3, ms2, hist2, counts3)

    k3 = pl.kernel(
        _sc_correction,
        out_type=jax.ShapeDtypeStruct((NW * L,), jnp.float32),
        mesh=mesh,
        compiler_params=sc_params,
        scratch_types=[
            pltpu.VMEM((CHUNK,), jnp.int32),
            pltpu.VMEM((CHUNK,), jnp.float32),
            pltpu.VMEM((3 * NB,), jnp.float32),
            pltpu.VMEM((3 * CHUNK,), jnp.int32),
            pltpu.VMEM((3 * CHUNK,), jnp.float32),
            pltpu.VMEM((3 * CHUNK,), jnp.float32),
            pltpu.VMEM((L,), jnp.float32),
            pltpu.SemaphoreType.DMA,
        ],
    )
    cpart = k3(tab.reshape(3 * NB), inv_img.reshape(P), lin_r, ints, rm, im)
    cpart = cpart.reshape(NW, L)

    loss = pl.pallas_call(
        _tc_finish,
        in_specs=[
            pl.BlockSpec((1, 1), lambda: (0, 0)),
            pl.BlockSpec((NW, L), lambda: (0, 0)),
        ],
        out_specs=pl.BlockSpec((1, 1), lambda: (0, 0)),
        out_shape=jax.ShapeDtypeStruct((1, 1), jnp.float32),
    )(s1, cpart)

    return loss.reshape(())


# speedup vs baseline: 432.1203x; 1.5648x over previous
import functools

import jax
import jax.numpy as jnp
from jax import lax
from jax.experimental import pallas as pl
from jax.experimental.pallas import tpu as pltpu
from jax.experimental.pallas import tpu_sc as plsc

H = 512
P = H * H
N = 65536
NC, NS, L = 2, 16, 16
NW = NC * NS
CHUNK = N // NW
NV = CHUNK // L
NB = 256
NSC = 6
HISTW = NSC * NB
SEG = P // NS
RB = H // 8
BLK = RB * H


def _dn255(x):
    return jnp.clip((x + 1.0) / 2.0, 0.0, 1.0) * 255.0


def _tc_prep(i0, i1, i2, f0, f1, f2, t0, t1, t2, ms, mt,
             im0_o, im1_o, im2_o, rm0_o, rm1_o, rm2_o, btp_o, s1_ref):
    r = pl.program_id(0)
    msv = ms[...]
    mtv = mt[...]
    part = jnp.zeros((1, 1), jnp.float32)
    bt = None
    for c, (inp, rf, tg, im_o, rm_o) in enumerate([
            (i0, f0, t0, im0_o, rm0_o),
            (i1, f1, t1, im1_o, rm1_o),
            (i2, f2, t2, im2_o, rm2_o)]):
        im = _dn255(inp[...]) * msv
        rm = _dn255(rf[...]) * msv
        tm = _dn255(tg[...]) * mtv
        im_o[...] = im.reshape(BLK)
        rm_o[...] = rm.reshape(BLK)
        btc = jnp.clip(tm, 0.0, 255.0).astype(jnp.int32)
        bt = btc if c == 0 else bt | (btc << (8 * c))
        part = part + jnp.sum(jnp.abs(im - rm)).reshape(1, 1)
    btp_o[...] = bt.reshape(BLK)

    @pl.when(r == 0)
    def _():
        s1_ref[...] = part

    @pl.when(r > 0)
    def _():
        s1_ref[...] += part


def _sc_gather_hist(rm0, rm1, rm2, btp, idx4,
                    hist_out, counts_out, lin_out, rmk_out,
                    idxr_v, idxt_v, tmp_v, g_v, bt_v,
                    hist16_v, histred_v, ones_v, cbuf_v,
                    rm0_sh, rm1_sh, rm2_sh, counts_sh, sem):
    cid = lax.axis_index("c")
    sid = lax.axis_index("s")
    wid = sid * NC + cid
    base = wid * CHUNK

    sl_st = pl.ds(sid * SEG, SEG)
    stages = [pltpu.async_copy(rm0.at[sl_st], rm0_sh.at[sl_st], sem),
              pltpu.async_copy(rm1.at[sl_st], rm1_sh.at[sl_st], sem),
              pltpu.async_copy(rm2.at[sl_st], rm2_sh.at[sl_st], sem)]

    pltpu.sync_copy(idx4.at[pl.ds(0 * N + base, CHUNK)], idxr_v)
    pltpu.sync_copy(idx4.at[pl.ds(1 * N + base, CHUNK)], tmp_v)

    def lstep(j, _):
        sl = pl.ds(j * L, L)
        idxr_v[sl] = idxr_v[sl] * H + tmp_v[sl]
        return 0
    lax.fori_loop(0, NV, lstep, 0)

    pltpu.sync_copy(idx4.at[pl.ds(2 * N + base, CHUNK)], idxt_v)
    pltpu.sync_copy(idx4.at[pl.ds(3 * N + base, CHUNK)], tmp_v)

    def tstep(j, _):
        sl = pl.ds(j * L, L)
        idxt_v[sl] = idxt_v[sl] * H + tmp_v[sl]
        return 0
    lax.fori_loop(0, NV, tstep, 0)

    dbt = pltpu.async_copy(btp.at[idxt_v], bt_v, sem)

    zero16 = jnp.zeros((L,), jnp.int32)
    one16 = jnp.full((L,), 1, jnp.int32)

    def zstep(j, _):
        hist16_v[pl.ds(j * L, L)] = zero16
        return 0
    lax.fori_loop(0, (L * HISTW) // L, zstep, 0)

    def onstep(j, _):
        ones_v[pl.ds(j * L, L)] = one16
        return 0
    lax.fori_loop(0, NV, onstep, 0)

    def cstep(j, _):
        cbuf_v[pl.ds(j * L, L)] = zero16
        return 0
    lax.fori_loop(0, SEG // L, cstep, 0)

    pltpu.sync_copy(cbuf_v, counts_sh.at[sl_st])
    for st in stages:
        st.wait()
    plsc.subcore_barrier()
    pltpu.sync_copy(ones_v, counts_sh.at[idxr_v], add=True)
    pltpu.sync_copy(idxr_v, lin_out.at[pl.ds(base, CHUNK)])

    descs = [pltpu.async_copy(rm0_sh.at[idxr_v], g_v.at[pl.ds(0, CHUNK)],
                              sem),
             pltpu.async_copy(rm1_sh.at[idxr_v],
                              g_v.at[pl.ds(CHUNK, CHUNK)], sem),
             pltpu.async_copy(rm2_sh.at[idxr_v],
                              g_v.at[pl.ds(2 * CHUNK, CHUNK)], sem)]
    for d in descs:
        d.wait()
    dbt.wait()

    laneoff = lax.iota(jnp.int32, L) * HISTW

    def step(i, _):
        off = i * L
        btp16 = bt_v[pl.ds(off, L)]
        for c in range(3):
            rmv = g_v[pl.ds(c * CHUNK + off, L)]
            b_r = jnp.clip(rmv, 0.0, 255.0).astype(jnp.int32)
            b_t = (btp16 >> (8 * c)) & 255
            plsc.addupdate_scatter(hist16_v, [laneoff + (c * NB + b_r)],
                                   one16)
            plsc.addupdate_scatter(hist16_v, [laneoff + ((3 + c) * NB + b_t)],
                                   one16)
        return 0
    lax.fori_loop(0, NV, step, 0)

    def rstep(j, _):
        acc = hist16_v[pl.ds(j * L, L)]
        for l in range(1, L):
            acc = acc + hist16_v[pl.ds(l * HISTW + j * L, L)]
        histred_v[pl.ds(j * L, L)] = acc
        return 0
    lax.fori_loop(0, HISTW // L, rstep, 0)

    pltpu.sync_copy(histred_v, hist_out.at[pl.ds(wid * HISTW, HISTW)])
    for c in range(3):
        pltpu.sync_copy(g_v.at[pl.ds(c * CHUNK, CHUNK)],
                        rmk_out.at[pl.ds(c * N + base, CHUNK)])

    plsc.subcore_barrier()
    pltpu.sync_copy(counts_sh.at[sl_st], cbuf_v)
    pltpu.sync_copy(cbuf_v, counts_out.at[pl.ds(cid * P + sid * SEG, SEG)])


def _tc_tables_inv(hist_ref, cnt0_ref, cnt1_ref, tab_ref, inv_ref):
    g = pl.program_id(0)

    @pl.when(g == 0)
    def _():
        acc = hist_ref[pl.ds(0, HISTW)]
        for w in range(1, NW):
            acc = acc + hist_ref[pl.ds(w * HISTW, HISTW)]
        hist6 = acc.reshape(NSC, NB).astype(jnp.float32)
        row = lax.broadcasted_iota(jnp.int32, (NB, NB), 0)
        col = lax.broadcasted_iota(jnp.int32, (NB, NB), 1)
        tri = (row <= col).astype(jnp.float32)
        cdf = jax.lax.dot(hist6, tri,
                          preferred_element_type=jnp.float32) * (1.0 / N)
        ii1 = lax.broadcasted_iota(jnp.int32, (NB,), 0).astype(jnp.float32)
        for c in range(3):
            rcdf = cdf[c]
            acdf = cdf[3 + c]
            cmp = (acdf[None, :] < rcdf[:, None]).astype(jnp.float32)
            t = jnp.sum(cmp, axis=1)
            tab = jnp.where(t >= 1.0, t,
                            jnp.where(rcdf >= acdf[0], 1.0, ii1))
            tab = jnp.where(ii1 == 0.0, 0.0,
                            jnp.where(ii1 == 255.0, 255.0, tab))
            tab_ref[pl.ds(c * NB, NB)] = tab

    cnt = cnt0_ref[...] + cnt1_ref[...]
    inv_ref[...] = 1.0 / cnt.astype(jnp.float32)


def _sc_correction(tab, inv_img, lin_r, rmk, im0, im1, im2,
                   cpart_out,
                   idxr_v, invc_v, tab_v, img_v, rmb_v, acc_v,
                   im0_sh, im1_sh, im2_sh, inv_sh, sem):
    cid = lax.axis_index("c")
    sid = lax.axis_index("s")
    wid = sid * NC + cid
    base = wid * CHUNK

    sl_st = pl.ds(sid * SEG, SEG)
    stages = [pltpu.async_copy(im0.at[sl_st], im0_sh.at[sl_st], sem),
              pltpu.async_copy(im1.at[sl_st], im1_sh.at[sl_st], sem),
              pltpu.async_copy(im2.at[sl_st], im2_sh.at[sl_st], sem),
              pltpu.async_copy(inv_img.at[sl_st], inv_sh.at[sl_st], sem)]

    pltpu.sync_copy(lin_r.at[pl.ds(base, CHUNK)], idxr_v)
    pltpu.sync_copy(tab, tab_v)
    for c in range(3):
        pltpu.sync_copy(rmk.at[pl.ds(c * N + base, CHUNK)],
                        rmb_v.at[pl.ds(c * CHUNK, CHUNK)])
    for st in stages:
        st.wait()
    plsc.subcore_barrier()

    descs = [pltpu.async_copy(inv_sh.at[idxr_v], invc_v, sem),
             pltpu.async_copy(im0_sh.at[idxr_v], img_v.at[pl.ds(0, CHUNK)],
                              sem),
             pltpu.async_copy(im1_sh.at[idxr_v],
                              img_v.at[pl.ds(CHUNK, CHUNK)], sem),
             pltpu.async_copy(im2_sh.at[idxr_v],
                              img_v.at[pl.ds(2 * CHUNK, CHUNK)], sem)]
    for d in descs:
        d.wait()

    def step(i, acc):
        off = i * L
        invv = invc_v[pl.ds(off, L)]
        for c in range(3):
            rmv = rmb_v[pl.ds(c * CHUNK + off, L)]
            imv = img_v[pl.ds(c * CHUNK + off, L)]
            b = jnp.clip(rmv, 0.0, 255.0).astype(jnp.int32)
            v = plsc.load_gather(tab_v, [b + (c * NB)])
            dlt = jnp.abs(imv - v) - jnp.abs(imv - rmv)
            acc = acc + dlt * invv
        return acc
    acc = lax.fori_loop(0, NV, step, jnp.zeros((L,), jnp.float32))
    acc_v[...] = acc
    pltpu.sync_copy(acc_v, cpart_out.at[pl.ds(wid * L, L)])


def _tc_finish(s1_ref, cpart_ref, out_ref):
    tot = s1_ref[...] + jnp.sum(cpart_ref[...]).reshape(1, 1)
    out_ref[...] = tot * (1.0 / (3.0 * P))


def kernel(input_data, target_data, mask_src, mask_tar, index, ref):
    inp2 = input_data.reshape(3 * H, H)
    tgt2 = target_data.reshape(3 * H, H)
    rf2 = ref.reshape(3 * H, H)
    ms2 = mask_src.reshape(H, H)
    mt2 = mask_tar.reshape(H, H)
    idx4 = index.reshape(4 * N).astype(jnp.int32)

    def chspec(c):
        return pl.BlockSpec((RB, H), lambda r, c=c: (c * 8 + r, 0))

    prep_out = pl.pallas_call(
        _tc_prep,
        grid=(8,),
        in_specs=[chspec(0), chspec(1), chspec(2),
                  chspec(0), chspec(1), chspec(2),
                  chspec(0), chspec(1), chspec(2),
                  pl.BlockSpec((RB, H), lambda r: (r, 0)),
                  pl.BlockSpec((RB, H), lambda r: (r, 0))],
        out_specs=[pl.BlockSpec((BLK,), lambda r: (r,))] * 7
        + [pl.BlockSpec((1, 1), lambda r: (0, 0))],
        out_shape=[jax.ShapeDtypeStruct((P,), jnp.float32)] * 6
        + [jax.ShapeDtypeStruct((P,), jnp.int32),
           jax.ShapeDtypeStruct((1, 1), jnp.float32)],
    )(inp2, inp2, inp2, rf2, rf2, rf2, tgt2, tgt2, tgt2, ms2, mt2)
    im0, im1, im2, rm0, rm1, rm2, btp, s1 = prep_out

    mesh = plsc.VectorSubcoreMesh(core_axis_name="c", subcore_axis_name="s",
                                  num_cores=NC, num_subcores=NS)
    sc_params = pltpu.CompilerParams(needs_layout_passes=False)

    k1 = pl.kernel(
        _sc_gather_hist,
        out_type=(
            jax.ShapeDtypeStruct((NW * HISTW,), jnp.int32),
            jax.ShapeDtypeStruct((NC * P,), jnp.int32),
            jax.ShapeDtypeStruct((N,), jnp.int32),
            jax.ShapeDtypeStruct((3 * N,), jnp.float32),
        ),
        mesh=mesh,
        compiler_params=sc_params,
        scratch_types=[
            pltpu.VMEM((CHUNK,), jnp.int32),
            pltpu.VMEM((CHUNK,), jnp.int32),
            pltpu.VMEM((CHUNK,), jnp.int32),
            pltpu.VMEM((3 * CHUNK,), jnp.float32),
            pltpu.VMEM((CHUNK,), jnp.int32),
            pltpu.VMEM((L * HISTW,), jnp.int32),
            pltpu.VMEM((HISTW,), jnp.int32),
            pltpu.VMEM((CHUNK,), jnp.int32),
            pltpu.VMEM((SEG,), jnp.int32),
            pltpu.VMEM_SHARED((P,), jnp.float32),
            pltpu.VMEM_SHARED((P,), jnp.float32),
            pltpu.VMEM_SHARED((P,), jnp.float32),
            pltpu.VMEM_SHARED((P,), jnp.int32),
            pltpu.SemaphoreType.DMA,
        ],
    )
    hist, counts, lin, rmk = k1(rm0, rm1, rm2, btp, idx4)

    steps = 8
    seg = P // steps
    tab, inv_img = pl.pallas_call(
        _tc_tables_inv,
        grid=(steps,),
        in_specs=[
            pl.BlockSpec((NW * HISTW,), lambda g: (0,)),
            pl.BlockSpec((seg,), lambda g: (g,)),
            pl.BlockSpec((seg,), lambda g: (g + steps,)),
        ],
        out_specs=[
            pl.BlockSpec((3 * NB,), lambda g: (0,)),
            pl.BlockSpec((seg,), lambda g: (g,)),
        ],
        out_shape=[
            jax.ShapeDtypeStruct((3 * NB,), jnp.float32),
            jax.ShapeDtypeStruct((P,), jnp.float32),
        ],
    )(hist, counts, counts)

    k3 = pl.kernel(
        _sc_correction,
        out_type=jax.ShapeDtypeStruct((NW * L,), jnp.float32),
        mesh=mesh,
        compiler_params=sc_params,
        scratch_types=[
            pltpu.VMEM((CHUNK,), jnp.int32),
            pltpu.VMEM((CHUNK,), jnp.float32),
            pltpu.VMEM((3 * NB,), jnp.float32),
            pltpu.VMEM((3 * CHUNK,), jnp.float32),
            pltpu.VMEM((3 * CHUNK,), jnp.float32),
            pltpu.VMEM((L,), jnp.float32),
            pltpu.VMEM_SHARED((P,), jnp.float32),
            pltpu.VMEM_SHARED((P,), jnp.float32),
            pltpu.VMEM_SHARED((P,), jnp.float32),
            pltpu.VMEM_SHARED((P,), jnp.float32),
            pltpu.SemaphoreType.DMA,
        ],
    )
    cpart = k3(tab, inv_img, lin, rmk, im0, im1, im2)

    loss = pl.pallas_call(
        _tc_finish,
        in_specs=[
            pl.BlockSpec((1, 1), lambda: (0, 0)),
            pl.BlockSpec((NW * L,), lambda: (0,)),
        ],
        out_specs=pl.BlockSpec((1, 1), lambda: (0, 0)),
        out_shape=jax.ShapeDtypeStruct((1, 1), jnp.float32),
    )(s1, cpart)

    return loss.reshape(())
